# Initial kernel scaffold; baseline (speedup 1.0000x reference)
import jax
import jax.numpy as jnp
from jax.experimental import pallas as pl


def kernel(x, table):
    raise NotImplementedError("write your pallas kernel here")


# speedup vs baseline: 2.8519x; 2.8519x over previous
import jax
import jax.numpy as jnp
from jax.experimental import pallas as pl


_SEQ_BLOCK = 512


def _add_kernel(x_ref, t_ref, o_ref):
    o_ref[...] = x_ref[...] + t_ref[...]


def kernel(x, table):
    batch, seq_len, d_model = x.shape
    n_seq = seq_len // _SEQ_BLOCK
    return pl.pallas_call(
        _add_kernel,
        grid=(n_seq, batch),
        in_specs=[
            pl.BlockSpec((1, _SEQ_BLOCK, d_model), lambda i, b: (b, i, 0)),
            pl.BlockSpec((_SEQ_BLOCK, d_model), lambda i, b: (i, 0)),
        ],
        out_specs=pl.BlockSpec((1, _SEQ_BLOCK, d_model), lambda i, b: (b, i, 0)),
        out_shape=jax.ShapeDtypeStruct(x.shape, x.dtype),
    )(x, table)


# speedup vs baseline: 3.1805x; 1.1152x over previous
import jax
import jax.numpy as jnp
from jax.experimental import pallas as pl


_SEQ_BLOCK = 1024


def _add_kernel(x_ref, t_ref, o_ref):
    o_ref[...] = x_ref[...] + t_ref[...]


def kernel(x, table):
    batch, seq_len, d_model = x.shape
    n_seq = seq_len // _SEQ_BLOCK
    return pl.pallas_call(
        _add_kernel,
        grid=(n_seq, batch),
        in_specs=[
            pl.BlockSpec((1, _SEQ_BLOCK, d_model), lambda i, b: (b, i, 0)),
            pl.BlockSpec((_SEQ_BLOCK, d_model), lambda i, b: (i, 0)),
        ],
        out_specs=pl.BlockSpec((1, _SEQ_BLOCK, d_model), lambda i, b: (b, i, 0)),
        out_shape=jax.ShapeDtypeStruct(x.shape, x.dtype),
    )(x, table)


# speedup vs baseline: 3.3116x; 1.0412x over previous
import jax
import jax.numpy as jnp
from jax.experimental import pallas as pl


_SEQ_BLOCK = 2048


def _add_kernel(x_ref, t_ref, o_ref):
    o_ref[...] = x_ref[...] + t_ref[...]


def kernel(x, table):
    batch, seq_len, d_model = x.shape
    n_seq = seq_len // _SEQ_BLOCK
    return pl.pallas_call(
        _add_kernel,
        grid=(n_seq, batch),
        in_specs=[
            pl.BlockSpec((1, _SEQ_BLOCK, d_model), lambda i, b: (b, i, 0)),
            pl.BlockSpec((_SEQ_BLOCK, d_model), lambda i, b: (i, 0)),
        ],
        out_specs=pl.BlockSpec((1, _SEQ_BLOCK, d_model), lambda i, b: (b, i, 0)),
        out_shape=jax.ShapeDtypeStruct(x.shape, x.dtype),
    )(x, table)


# speedup vs baseline: 3.3128x; 1.0004x over previous
import jax
import jax.numpy as jnp
from jax.experimental import pallas as pl
from jax.experimental.pallas import tpu as pltpu


_SEQ_BLOCK = 2048


def _add_kernel(x_ref, t_ref, o_ref):
    o_ref[...] = x_ref[...] + t_ref[...]


def kernel(x, table):
    batch, seq_len, d_model = x.shape
    n_seq = seq_len // _SEQ_BLOCK
    return pl.pallas_call(
        _add_kernel,
        grid=(n_seq, batch),
        in_specs=[
            pl.BlockSpec((1, _SEQ_BLOCK, d_model), lambda i, b: (b, i, 0)),
            pl.BlockSpec((_SEQ_BLOCK, d_model), lambda i, b: (i, 0)),
        ],
        out_specs=pl.BlockSpec((1, _SEQ_BLOCK, d_model), lambda i, b: (b, i, 0)),
        out_shape=jax.ShapeDtypeStruct(x.shape, x.dtype),
        compiler_params=pltpu.CompilerParams(
            dimension_semantics=("parallel", "parallel"),
        ),
    )(x, table)
